# baseline (speedup 1.0000x reference)
import functools

import jax
import jax.numpy as jnp
from jax import lax
from jax.experimental import pallas as pl
from jax.experimental.pallas import tpu as pltpu
from jax.experimental.pallas import tpu_sc as plsc

NQ = 1024
DIM = 128
KREAL = 100000
TOPN = 20
CHUNK = 128
KB = 2048
NKB = -(-KREAL // KB)
CPB = KB // CHUNK
C = NKB * CPB
NEG = -3.0e38
IMAX = 0x7FFFFFFF

NW = 32
SUB = 128
B = NQ * TOPN
BPW = B // NW
NSUB = BPW // SUB
QPW = NQ // NW


def _argmax_iter(x, g):
    m = jnp.max(x, axis=1, keepdims=True)
    sel = jnp.min(jnp.where(x == m, g, IMAX), axis=1, keepdims=True)
    return m, sel


def _mm_body(q_ref, k_ref, s3_ref, m_ref):
    i = pl.program_id(0)
    q = q_ref[...]
    n = jnp.sqrt(jnp.sum(q * q, axis=1, keepdims=True))
    qn = q / jnp.maximum(n, 1e-12)
    s = lax.dot_general(qn, k_ref[...], (((1,), (1,)), ((), ())),
                        preferred_element_type=jnp.float32)

    def _emit(sv):
        ms = []
        for c in range(CPB):
            blk = sv[:, c * CHUNK:(c + 1) * CHUNK]
            s3_ref[c] = blk
            ms.append(jnp.max(blk, axis=1, keepdims=True))
        m_ref[0] = jnp.concatenate(ms, axis=1)

    @pl.when(i < NKB - 1)
    def _steady():
        _emit(s)

    @pl.when(i == NKB - 1)
    def _last():
        col = i * KB + lax.broadcasted_iota(jnp.int32, s.shape, 1)
        _emit(jnp.where(col < KREAL, s, NEG))


def _matmul(query, queue_k, interpret=False):
    return pl.pallas_call(
        _mm_body,
        grid=(NKB,),
        in_specs=[
            pl.BlockSpec((NQ, DIM), lambda i: (0, 0)),
            pl.BlockSpec((KB, DIM), lambda i: (i, 0)),
        ],
        out_specs=[
            pl.BlockSpec((CPB, NQ, CHUNK), lambda i: (i, 0, 0)),
            pl.BlockSpec((1, NQ, CPB), lambda i: (i, 0, 0)),
        ],
        out_shape=[
            jax.ShapeDtypeStruct((C, NQ, CHUNK), jnp.float32),
            jax.ShapeDtypeStruct((NKB, NQ, CPB), jnp.float32),
        ],
        compiler_params=pltpu.CompilerParams(
            dimension_semantics=("arbitrary",)),
        interpret=interpret,
    )(query, queue_k)


def _select_chunks_body(m_ref, topc_ref, rowidt_ref):
    x = jnp.concatenate([m_ref[j] for j in range(NKB)], axis=1)
    g = lax.broadcasted_iota(jnp.int32, x.shape, 1)
    qid = lax.broadcasted_iota(jnp.int32, (NQ, 1), 0)
    tcs, rids = [], []
    for _ in range(TOPN):
        _, sel = _argmax_iter(x, g)
        tcs.append(sel)
        rids.append(sel * NQ + qid)
        x = jnp.where(g == sel, NEG, x)
    topc_ref[...] = jnp.concatenate(tcs, axis=1)
    rt = jnp.transpose(jnp.concatenate(rids, axis=1))
    for t in range(TOPN):
        rowidt_ref[pl.ds(t * NQ, NQ)] = rt[t]


def _select_chunks(m, interpret=False):
    return pl.pallas_call(
        _select_chunks_body,
        out_shape=[
            jax.ShapeDtypeStruct((NQ, TOPN), jnp.int32),
            jax.ShapeDtypeStruct((B,), jnp.int32),
        ],
        interpret=interpret,
    )(m)


def _select_final_body(cand_ref, topc_ref, topi2_ref):
    tc = topc_ref[...]
    off = lax.broadcasted_iota(jnp.int32, (NQ, CHUNK), 1)
    gs = [tc[:, j:j + 1] * CHUNK + off for j in range(TOPN)]
    xs = [cand_ref[j] for j in range(TOPN)]
    sels = []
    for _ in range(TOPN):
        mm = xs[0]
        for j in range(1, TOPN):
            mm = jnp.maximum(mm, xs[j])
        m = jnp.max(mm, axis=1, keepdims=True)
        cm = jnp.where(xs[0] == m, gs[0], IMAX)
        for j in range(1, TOPN):
            cm = jnp.minimum(cm, jnp.where(xs[j] == m, gs[j], IMAX))
        sel = jnp.min(cm, axis=1, keepdims=True)
        sels.append(sel)
        for j in range(TOPN):
            xs[j] = jnp.where(gs[j] == sel, NEG, xs[j])
    rt = jnp.transpose(jnp.concatenate(sels, axis=1))
    for t in range(TOPN):
        topi2_ref[pl.ds(t * NQ, NQ)] = rt[t]


def _select_final(cand3, topc, interpret=False):
    return pl.pallas_call(
        _select_final_body,
        out_shape=jax.ShapeDtypeStruct((B,), jnp.int32),
        interpret=interpret,
    )(cand3, topc)


def _make_sc_gather(n_tables):
    mesh = plsc.VectorSubcoreMesh(
        core_axis_name="c", subcore_axis_name="s",
        num_cores=2, num_subcores=16)
    scratch = [pltpu.VMEM((SUB, DIM), jnp.float32) for _ in range(n_tables)]
    scratch += [pltpu.VMEM((SUB,), jnp.int32)]
    scratch += [pltpu.SemaphoreType.DMA for _ in range(2 * n_tables)]

    @functools.partial(
        pl.kernel,
        out_type=[jax.ShapeDtypeStruct((B, DIM), jnp.float32)
                  for _ in range(n_tables)],
        mesh=mesh,
        scratch_types=scratch,
    )
    def gather(*refs):
        tables = refs[:n_tables]
        idx_hbm = refs[n_tables]
        outs = refs[n_tables + 1:2 * n_tables + 1]
        bufs = refs[2 * n_tables + 1:3 * n_tables + 1]
        idx_v = refs[3 * n_tables + 1]
        gsems = refs[3 * n_tables + 2:4 * n_tables + 2]
        wsems = refs[4 * n_tables + 2:]
        wid = lax.axis_index("s") * 2 + lax.axis_index("c")
        base = wid * BPW
        for j in range(NSUB):
            off = base + j * SUB
            pltpu.sync_copy(idx_hbm.at[pl.ds(off, SUB)], idx_v)
            for t in range(n_tables):
                if j > 0:
                    pltpu.make_async_copy(
                        bufs[t], outs[t].at[pl.ds(base, SUB)],
                        wsems[t]).wait()
                pltpu.async_copy(tables[t].at[idx_v], bufs[t], gsems[t])
            for t in range(n_tables):
                pltpu.make_async_copy(
                    tables[t].at[idx_v], bufs[t], gsems[t]).wait()
                pltpu.async_copy(bufs[t], outs[t].at[pl.ds(off, SUB)],
                                 wsems[t])
        for t in range(n_tables):
            pltpu.make_async_copy(bufs[t], outs[t].at[pl.ds(base, SUB)],
                                  wsems[t]).wait()

    return gather


QB = 128


def _xpose_body(ak_ref, av_ref, ok_ref, ov_ref):
    for a, o in ((ak_ref, ok_ref), (av_ref, ov_ref)):
        for t in range(TOPN):
            o[:, t:t + 1, :] = a[t][:, None, :]


def _xpose(gk3, gv3, interpret=False):
    return pl.pallas_call(
        _xpose_body,
        grid=(NQ // QB,),
        in_specs=[
            pl.BlockSpec((TOPN, QB, DIM), lambda i: (0, i, 0)),
            pl.BlockSpec((TOPN, QB, DIM), lambda i: (0, i, 0)),
        ],
        out_specs=[
            pl.BlockSpec((QB, TOPN, DIM), lambda i: (i, 0, 0)),
            pl.BlockSpec((QB, TOPN, DIM), lambda i: (i, 0, 0)),
        ],
        out_shape=[
            jax.ShapeDtypeStruct((NQ, TOPN, DIM), jnp.float32),
            jax.ShapeDtypeStruct((NQ, TOPN, DIM), jnp.float32),
        ],
        compiler_params=pltpu.CompilerParams(
            dimension_semantics=("arbitrary",)),
        interpret=interpret,
    )(gk3, gv3)


def _kernel_impl(query, queue_k, queue_v, interpret=False):
    s3, m = _matmul(query, queue_k, interpret=interpret)
    topc, rowidt = _select_chunks(m, interpret=interpret)
    if interpret:
        cand = jnp.take(s3.reshape(C * NQ, CHUNK), rowidt, axis=0)
    else:
        (cand,) = _make_sc_gather(1)(s3.reshape(C * NQ, CHUNK), rowidt)
    topit = _select_final(cand.reshape(TOPN, NQ, CHUNK), topc,
                          interpret=interpret)
    if interpret:
        gk2 = jnp.take(queue_k, topit, axis=0)
        gv2 = jnp.take(queue_v, topit, axis=0)
    else:
        gk2, gv2 = _make_sc_gather(2)(queue_k, queue_v, topit)
    return _xpose(gk2.reshape(TOPN, NQ, DIM), gv2.reshape(TOPN, NQ, DIM),
                  interpret=interpret)


def kernel(query, queue_k, queue_v):
    return _kernel_impl(query, queue_k, queue_v)


# speedup vs baseline: 1.2061x; 1.2061x over previous
import functools

import jax
import jax.numpy as jnp
from jax import lax
from jax.experimental import pallas as pl
from jax.experimental.pallas import tpu as pltpu
from jax.experimental.pallas import tpu_sc as plsc

NQ = 1024
DIM = 128
KREAL = 100000
TOPN = 20
CHUNK = 128
KB = 2048
NKB = -(-KREAL // KB)
CPB = KB // CHUNK
C = NKB * CPB
NEG = -3.0e38
IMAX = 0x7FFFFFFF

NW = 32
SUB = 128
B = NQ * TOPN
BPW = B // NW
NSUB = BPW // SUB
QPW = NQ // NW


def _argmax_iter(x, g):
    m = jnp.max(x, axis=1, keepdims=True)
    sel = jnp.min(jnp.where(x == m, g, IMAX), axis=1, keepdims=True)
    return m, sel


def _mm_body(q_ref, k_ref, s3_ref, m_ref):
    i = pl.program_id(0)
    q = q_ref[...]
    n = jnp.sqrt(jnp.sum(q * q, axis=1, keepdims=True))
    qn = q / jnp.maximum(n, 1e-12)
    s = lax.dot_general(qn, k_ref[...], (((1,), (1,)), ((), ())),
                        preferred_element_type=jnp.float32)
    col = i * KB + lax.broadcasted_iota(jnp.int32, s.shape, 1)
    s = jnp.where(col < KREAL, s, NEG)
    ms = []
    for c in range(CPB):
        blk = s[:, c * CHUNK:(c + 1) * CHUNK]
        s3_ref[c] = blk
        ms.append(jnp.max(blk, axis=1, keepdims=True))
    m_ref[0] = jnp.concatenate(ms, axis=1)


def _matmul(query, queue_k, interpret=False):
    return pl.pallas_call(
        _mm_body,
        grid=(NKB,),
        in_specs=[
            pl.BlockSpec((NQ, DIM), lambda i: (0, 0)),
            pl.BlockSpec((KB, DIM), lambda i: (i, 0)),
        ],
        out_specs=[
            pl.BlockSpec((CPB, NQ, CHUNK), lambda i: (i, 0, 0)),
            pl.BlockSpec((1, NQ, CPB), lambda i: (i, 0, 0)),
        ],
        out_shape=[
            jax.ShapeDtypeStruct((C, NQ, CHUNK), jnp.float32),
            jax.ShapeDtypeStruct((NKB, NQ, CPB), jnp.float32),
        ],
        compiler_params=pltpu.CompilerParams(
            dimension_semantics=("arbitrary",)),
        interpret=interpret,
    )(query, queue_k)


def _select_chunks_body(m_ref, topc_ref, rowidt_ref):
    x = jnp.concatenate([m_ref[j] for j in range(NKB)], axis=1)
    g = lax.broadcasted_iota(jnp.int32, x.shape, 1)
    qid = lax.broadcasted_iota(jnp.int32, (NQ, 1), 0)
    tcs, rids = [], []
    for _ in range(TOPN):
        _, sel = _argmax_iter(x, g)
        tcs.append(sel)
        rids.append(sel * NQ + qid)
        x = jnp.where(g == sel, NEG, x)
    topc_ref[...] = jnp.concatenate(tcs, axis=1)
    rt = jnp.transpose(jnp.concatenate(rids, axis=1))
    for t in range(TOPN):
        rowidt_ref[pl.ds(t * NQ, NQ)] = rt[t]


def _select_chunks(m, interpret=False):
    return pl.pallas_call(
        _select_chunks_body,
        out_shape=[
            jax.ShapeDtypeStruct((NQ, TOPN), jnp.int32),
            jax.ShapeDtypeStruct((B,), jnp.int32),
        ],
        interpret=interpret,
    )(m)


def _select_final_body(cand_ref, topc_ref, topi2_ref):
    tc = topc_ref[...]
    off = lax.broadcasted_iota(jnp.int32, (NQ, CHUNK), 1)
    gs = [tc[:, j:j + 1] * CHUNK + off for j in range(TOPN)]
    xs = [cand_ref[j] for j in range(TOPN)]
    sels = []
    for _ in range(TOPN):
        mm = xs[0]
        for j in range(1, TOPN):
            mm = jnp.maximum(mm, xs[j])
        m = jnp.max(mm, axis=1, keepdims=True)
        cm = jnp.where(xs[0] == m, gs[0], IMAX)
        for j in range(1, TOPN):
            cm = jnp.minimum(cm, jnp.where(xs[j] == m, gs[j], IMAX))
        sel = jnp.min(cm, axis=1, keepdims=True)
        sels.append(sel)
        for j in range(TOPN):
            xs[j] = jnp.where(gs[j] == sel, NEG, xs[j])
    rt = jnp.transpose(jnp.concatenate(sels, axis=1))
    for t in range(TOPN):
        topi2_ref[pl.ds(t * NQ, NQ)] = rt[t]


def _select_final(cand3, topc, interpret=False):
    return pl.pallas_call(
        _select_final_body,
        out_shape=jax.ShapeDtypeStruct((B,), jnp.int32),
        interpret=interpret,
    )(cand3, topc)


def _make_sc_gather(n_tables):
    mesh = plsc.VectorSubcoreMesh(
        core_axis_name="c", subcore_axis_name="s",
        num_cores=2, num_subcores=16)
    scratch = [pltpu.VMEM((SUB, DIM), jnp.float32) for _ in range(n_tables)]
    scratch += [pltpu.VMEM((SUB,), jnp.int32)]
    scratch += [pltpu.SemaphoreType.DMA for _ in range(2 * n_tables)]

    @functools.partial(
        pl.kernel,
        out_type=[jax.ShapeDtypeStruct((B, DIM), jnp.float32)
                  for _ in range(n_tables)],
        mesh=mesh,
        scratch_types=scratch,
    )
    def gather(*refs):
        tables = refs[:n_tables]
        idx_hbm = refs[n_tables]
        outs = refs[n_tables + 1:2 * n_tables + 1]
        bufs = refs[2 * n_tables + 1:3 * n_tables + 1]
        idx_v = refs[3 * n_tables + 1]
        gsems = refs[3 * n_tables + 2:4 * n_tables + 2]
        wsems = refs[4 * n_tables + 2:]
        wid = lax.axis_index("s") * 2 + lax.axis_index("c")
        base = wid * BPW
        for j in range(NSUB):
            off = base + j * SUB
            pltpu.sync_copy(idx_hbm.at[pl.ds(off, SUB)], idx_v)
            for t in range(n_tables):
                if j > 0:
                    pltpu.make_async_copy(
                        bufs[t], outs[t].at[pl.ds(base, SUB)],
                        wsems[t]).wait()
                pltpu.async_copy(tables[t].at[idx_v], bufs[t], gsems[t])
            for t in range(n_tables):
                pltpu.make_async_copy(
                    tables[t].at[idx_v], bufs[t], gsems[t]).wait()
                pltpu.async_copy(bufs[t], outs[t].at[pl.ds(off, SUB)],
                                 wsems[t])
        for t in range(n_tables):
            pltpu.make_async_copy(bufs[t], outs[t].at[pl.ds(base, SUB)],
                                  wsems[t]).wait()

    return gather


QB = 128


def _xpose_body(ak_ref, av_ref, ok_ref, ov_ref):
    for a, o in ((ak_ref, ok_ref), (av_ref, ov_ref)):
        for t in range(TOPN):
            o[:, t:t + 1, :] = a[t][:, None, :]


def _xpose(gk3, gv3, interpret=False):
    return pl.pallas_call(
        _xpose_body,
        grid=(NQ // QB,),
        in_specs=[
            pl.BlockSpec((TOPN, QB, DIM), lambda i: (0, i, 0)),
            pl.BlockSpec((TOPN, QB, DIM), lambda i: (0, i, 0)),
        ],
        out_specs=[
            pl.BlockSpec((QB, TOPN, DIM), lambda i: (i, 0, 0)),
            pl.BlockSpec((QB, TOPN, DIM), lambda i: (i, 0, 0)),
        ],
        out_shape=[
            jax.ShapeDtypeStruct((NQ, TOPN, DIM), jnp.float32),
            jax.ShapeDtypeStruct((NQ, TOPN, DIM), jnp.float32),
        ],
        compiler_params=pltpu.CompilerParams(
            dimension_semantics=("arbitrary",)),
        interpret=interpret,
    )(gk3, gv3)


def _kernel_impl(query, queue_k, queue_v, interpret=False):
    s3, m = _matmul(query, queue_k, interpret=interpret)
    topc, rowidt = _select_chunks(m, interpret=interpret)
    if interpret:
        cand = jnp.take(s3.reshape(C * NQ, CHUNK), rowidt, axis=0)
    else:
        (cand,) = _make_sc_gather(1)(s3.reshape(C * NQ, CHUNK), rowidt)
    topit = _select_final(cand.reshape(TOPN, NQ, CHUNK), topc,
                          interpret=interpret)
    if interpret:
        gk2 = jnp.take(queue_k, topit, axis=0)
        gv2 = jnp.take(queue_v, topit, axis=0)
    else:
        gk2, gv2 = _make_sc_gather(2)(queue_k, queue_v, topit)
    return _xpose(gk2.reshape(TOPN, NQ, DIM), gv2.reshape(TOPN, NQ, DIM),
                  interpret=interpret)


def kernel(query, queue_k, queue_v):
    return _kernel_impl(query, queue_k, queue_v)


# speedup vs baseline: 1.2065x; 1.0003x over previous
import functools

import jax
import jax.numpy as jnp
from jax import lax
from jax.experimental import pallas as pl
from jax.experimental.pallas import tpu as pltpu
from jax.experimental.pallas import tpu_sc as plsc

NQ = 1024
DIM = 128
KREAL = 100000
TOPN = 20
CHUNK = 128
KB = 2048
NKB = -(-KREAL // KB)
CPB = KB // CHUNK
C = NKB * CPB
NEG = -3.0e38
IMAX = 0x7FFFFFFF

NW = 32
SUB = 128
B = NQ * TOPN
BPW = B // NW
NSUB = BPW // SUB


def _argmax_iter(x, g):
    m = jnp.max(x, axis=1, keepdims=True)
    sel = jnp.min(jnp.where(x == m, g, IMAX), axis=1, keepdims=True)
    return m, sel


def _mm_body(q_ref, k_ref, s3_ref, m_ref):
    i = pl.program_id(0)
    q = q_ref[...]
    n = jnp.sqrt(jnp.sum(q * q, axis=1, keepdims=True))
    qn = q / jnp.maximum(n, 1e-12)
    s = lax.dot_general(qn, k_ref[...], (((1,), (1,)), ((), ())),
                        preferred_element_type=jnp.float32)
    col = i * KB + lax.broadcasted_iota(jnp.int32, s.shape, 1)
    s = jnp.where(col < KREAL, s, NEG)
    ms = []
    for c in range(CPB):
        blk = s[:, c * CHUNK:(c + 1) * CHUNK]
        s3_ref[c] = blk
        ms.append(jnp.max(blk, axis=1, keepdims=True))
    m_ref[0] = jnp.concatenate(ms, axis=1)


def _matmul(query, queue_k, interpret=False):
    return pl.pallas_call(
        _mm_body,
        grid=(NKB,),
        in_specs=[
            pl.BlockSpec((NQ, DIM), lambda i: (0, 0)),
            pl.BlockSpec((KB, DIM), lambda i: (i, 0)),
        ],
        out_specs=[
            pl.BlockSpec((CPB, NQ, CHUNK), lambda i: (i, 0, 0)),
            pl.BlockSpec((1, NQ, CPB), lambda i: (i, 0, 0)),
        ],
        out_shape=[
            jax.ShapeDtypeStruct((C, NQ, CHUNK), jnp.float32),
            jax.ShapeDtypeStruct((NKB, NQ, CPB), jnp.float32),
        ],
        compiler_params=pltpu.CompilerParams(
            dimension_semantics=("arbitrary",)),
        interpret=interpret,
    )(query, queue_k)


def _select_chunks_body(m_ref, topc_ref, rowidt_ref):
    x = jnp.concatenate([m_ref[j] for j in range(NKB)], axis=1)
    g = lax.broadcasted_iota(jnp.int32, x.shape, 1)
    qid = lax.broadcasted_iota(jnp.int32, (NQ, 1), 0)
    tcs, rids = [], []
    for _ in range(TOPN):
        _, sel = _argmax_iter(x, g)
        tcs.append(sel)
        rids.append(sel * NQ + qid)
        x = jnp.where(g == sel, NEG, x)
    topc_ref[...] = jnp.concatenate(tcs, axis=1)
    rt = jnp.transpose(jnp.concatenate(rids, axis=1))
    for t in range(TOPN):
        rowidt_ref[pl.ds(t * NQ, NQ)] = rt[t]


def _select_chunks(m, interpret=False):
    return pl.pallas_call(
        _select_chunks_body,
        out_shape=[
            jax.ShapeDtypeStruct((NQ, TOPN), jnp.int32),
            jax.ShapeDtypeStruct((B,), jnp.int32),
        ],
        interpret=interpret,
    )(m)


def _select_final_body(cand_ref, topc_ref, topi2_ref):
    tc = topc_ref[...]
    off = lax.broadcasted_iota(jnp.int32, (NQ, CHUNK), 1)
    gs = [tc[:, j:j + 1] * CHUNK + off for j in range(TOPN)]
    xs = [cand_ref[j] for j in range(TOPN)]
    sels = []
    for _ in range(TOPN):
        mm = xs[0]
        for j in range(1, TOPN):
            mm = jnp.maximum(mm, xs[j])
        m = jnp.max(mm, axis=1, keepdims=True)
        cm = jnp.where(xs[0] == m, gs[0], IMAX)
        for j in range(1, TOPN):
            cm = jnp.minimum(cm, jnp.where(xs[j] == m, gs[j], IMAX))
        sel = jnp.min(cm, axis=1, keepdims=True)
        sels.append(sel)
        for j in range(TOPN):
            xs[j] = jnp.where(gs[j] == sel, NEG, xs[j])
    rt = jnp.transpose(jnp.concatenate(sels, axis=1))
    for t in range(TOPN):
        topi2_ref[pl.ds(t * NQ, NQ)] = rt[t]


def _select_final(cand3, topc, interpret=False):
    return pl.pallas_call(
        _select_final_body,
        out_shape=jax.ShapeDtypeStruct((B,), jnp.int32),
        interpret=interpret,
    )(cand3, topc)


def _make_sc_gather(n_tables):
    mesh = plsc.VectorSubcoreMesh(
        core_axis_name="c", subcore_axis_name="s",
        num_cores=2, num_subcores=16)
    scratch = [pltpu.VMEM((SUB, DIM), jnp.float32) for _ in range(n_tables)]
    scratch += [pltpu.VMEM((SUB,), jnp.int32)]
    scratch += [pltpu.SemaphoreType.DMA for _ in range(2 * n_tables)]

    @functools.partial(
        pl.kernel,
        out_type=[jax.ShapeDtypeStruct((B, DIM), jnp.float32)
                  for _ in range(n_tables)],
        mesh=mesh,
        scratch_types=scratch,
    )
    def gather(*refs):
        tables = refs[:n_tables]
        idx_hbm = refs[n_tables]
        outs = refs[n_tables + 1:2 * n_tables + 1]
        bufs = refs[2 * n_tables + 1:3 * n_tables + 1]
        idx_v = refs[3 * n_tables + 1]
        gsems = refs[3 * n_tables + 2:4 * n_tables + 2]
        wsems = refs[4 * n_tables + 2:]
        wid = lax.axis_index("s") * 2 + lax.axis_index("c")
        base = wid * BPW
        for j in range(NSUB):
            off = base + j * SUB
            pltpu.sync_copy(idx_hbm.at[pl.ds(off, SUB)], idx_v)
            for t in range(n_tables):
                if j > 0:
                    pltpu.make_async_copy(
                        bufs[t], outs[t].at[pl.ds(base, SUB)],
                        wsems[t]).wait()
                pltpu.async_copy(tables[t].at[idx_v], bufs[t], gsems[t])
            for t in range(n_tables):
                pltpu.make_async_copy(
                    tables[t].at[idx_v], bufs[t], gsems[t]).wait()
                pltpu.async_copy(bufs[t], outs[t].at[pl.ds(off, SUB)],
                                 wsems[t])
        for t in range(n_tables):
            pltpu.make_async_copy(bufs[t], outs[t].at[pl.ds(base, SUB)],
                                  wsems[t]).wait()

    return gather


QB = 128


def _xpose_body(ak_ref, av_ref, ok_ref, ov_ref):
    for a, o in ((ak_ref, ok_ref), (av_ref, ov_ref)):
        for t in range(TOPN):
            o[:, t:t + 1, :] = a[t][:, None, :]


def _xpose(gk3, gv3, interpret=False):
    return pl.pallas_call(
        _xpose_body,
        grid=(NQ // QB,),
        in_specs=[
            pl.BlockSpec((TOPN, QB, DIM), lambda i: (0, i, 0)),
            pl.BlockSpec((TOPN, QB, DIM), lambda i: (0, i, 0)),
        ],
        out_specs=[
            pl.BlockSpec((QB, TOPN, DIM), lambda i: (i, 0, 0)),
            pl.BlockSpec((QB, TOPN, DIM), lambda i: (i, 0, 0)),
        ],
        out_shape=[
            jax.ShapeDtypeStruct((NQ, TOPN, DIM), jnp.float32),
            jax.ShapeDtypeStruct((NQ, TOPN, DIM), jnp.float32),
        ],
        compiler_params=pltpu.CompilerParams(
            dimension_semantics=("arbitrary",)),
        interpret=interpret,
    )(gk3, gv3)


def _kernel_impl(query, queue_k, queue_v, interpret=False):
    s3, m = _matmul(query, queue_k, interpret=interpret)
    topc, rowidt = _select_chunks(m, interpret=interpret)
    if interpret:
        cand = jnp.take(s3.reshape(C * NQ, CHUNK), rowidt, axis=0)
    else:
        (cand,) = _make_sc_gather(1)(s3.reshape(C * NQ, CHUNK), rowidt)
    topit = _select_final(cand.reshape(TOPN, NQ, CHUNK), topc,
                          interpret=interpret)
    if interpret:
        gk2 = jnp.take(queue_k, topit, axis=0)
        gv2 = jnp.take(queue_v, topit, axis=0)
    else:
        gk2, gv2 = _make_sc_gather(2)(queue_k, queue_v, topit)
    return _xpose(gk2.reshape(TOPN, NQ, DIM), gv2.reshape(TOPN, NQ, DIM),
                  interpret=interpret)


def kernel(query, queue_k, queue_v):
    return _kernel_impl(query, queue_k, queue_v)
